# baseline (speedup 1.0000x reference)
import functools

import jax
import jax.numpy as jnp
from jax import lax
from jax.experimental import pallas as pl
from jax.experimental.pallas import tpu as pltpu
from jax.experimental.pallas import tpu_sc as plsc

NC, NS = 2, 16
NW = NC * NS
B = 16384
D = 64
BPW = B // NW


def _gather_body(u_hbm, i_hbm, ut_hbm, it_hbm, ue_out, ie_out,
                 uidx, iidx, usem, isem):
    wid = lax.axis_index("s") * NC + lax.axis_index("c")
    base = wid * BPW
    pltpu.sync_copy(u_hbm.at[pl.ds(base, BPW)], uidx)
    pltpu.sync_copy(i_hbm.at[pl.ds(base, BPW)], iidx)

    def issue(jj, _):
        b = jj * 16
        vu = uidx[pl.ds(b, 16)]
        vi = iidx[pl.ds(b, 16)]
        for k in range(16):
            n = base + b + k
            pltpu.async_copy(ut_hbm.at[pl.ds(vu[k], 1)],
                             ue_out.at[pl.ds(n, 1)], usem)
            pltpu.async_copy(it_hbm.at[pl.ds(vi[k], 1)],
                             ie_out.at[pl.ds(n, 1)], isem)
        return _

    lax.fori_loop(0, BPW // 16, issue, 0)

    def drain(j, _):
        pltpu.make_async_copy(ut_hbm.at[pl.ds(0, 1)],
                              ue_out.at[pl.ds(base, 1)], usem).wait()
        pltpu.make_async_copy(it_hbm.at[pl.ds(0, 1)],
                              ie_out.at[pl.ds(base, 1)], isem).wait()
        return _

    lax.fori_loop(0, BPW, drain, 0)


def _make_gather():
    mesh = plsc.VectorSubcoreMesh(core_axis_name="c", subcore_axis_name="s")
    return pl.kernel(
        _gather_body,
        out_type=(
            jax.ShapeDtypeStruct((B, D), jnp.float32),
            jax.ShapeDtypeStruct((B, D), jnp.float32),
        ),
        mesh=mesh,
        scratch_types=[
            pltpu.VMEM((BPW,), jnp.int32),
            pltpu.VMEM((BPW,), jnp.int32),
            pltpu.SemaphoreType.DMA,
            pltpu.SemaphoreType.DMA,
        ],
    )


BLK = 2048


def _mlp_body(ue, ie, w1a, w1b, b1, w2, b2, w3, b3, out):
    h = jnp.dot(ue[...], w1a[...], preferred_element_type=jnp.float32)
    h = h + jnp.dot(ie[...], w1b[...], preferred_element_type=jnp.float32)
    h = jnp.maximum(h + b1[...], 0.0)
    h = jnp.dot(h, w2[...], preferred_element_type=jnp.float32) + b2[...]
    h = jnp.maximum(h, 0.0)
    o = jnp.dot(h, w3[...], preferred_element_type=jnp.float32) + b3[...]
    out[...] = jax.nn.sigmoid(o)


def _mlp(ue, ie, W1a, W1b, b1, W2, b2, W3, b3):
    grid = (B // BLK,)
    full = lambda g: (0, 0)
    return pl.pallas_call(
        _mlp_body,
        grid=grid,
        in_specs=[
            pl.BlockSpec((BLK, D), lambda g: (g, 0)),
            pl.BlockSpec((BLK, D), lambda g: (g, 0)),
            pl.BlockSpec(W1a.shape, full),
            pl.BlockSpec(W1b.shape, full),
            pl.BlockSpec(b1.shape, full),
            pl.BlockSpec(W2.shape, full),
            pl.BlockSpec(b2.shape, full),
            pl.BlockSpec(W3.shape, full),
            pl.BlockSpec(b3.shape, full),
        ],
        out_specs=pl.BlockSpec((BLK, 1), lambda g: (g, 0)),
        out_shape=jax.ShapeDtypeStruct((B, 1), jnp.float32),
    )(ue, ie, W1a, W1b, b1, W2, b2, W3, b3)


def kernel(u, i, user_table, item_table, W1, b1, W2, b2, W3, b3):
    ue, ie = _make_gather()(u, i, user_table, item_table)
    out = _mlp(ue, ie,
               W1[:D], W1[D:],
               b1.reshape(1, -1),
               W2, b2.reshape(1, -1),
               W3, b3.reshape(1, -1))
    return jnp.squeeze(out, axis=-1)


# speedup vs baseline: 1.0724x; 1.0724x over previous
import functools

import jax
import jax.numpy as jnp
from jax import lax
from jax.experimental import pallas as pl
from jax.experimental.pallas import tpu as pltpu
from jax.experimental.pallas import tpu_sc as plsc

NC, NS = 2, 16
NW = NC * NS
B = 16384
D = 64
BPW = B // NW


CH = 128
NCH = BPW // CH


def _gather_body(u_hbm, i_hbm, ut_hbm, it_hbm, ue_out, ie_out,
                 uidx, iidx, urows, irows, usem, isem):
    wid = lax.axis_index("s") * NC + lax.axis_index("c")
    base = wid * BPW
    pltpu.sync_copy(u_hbm.at[pl.ds(base, BPW)], uidx)
    pltpu.sync_copy(i_hbm.at[pl.ds(base, BPW)], iidx)
    copies = []
    for j in range(NCH):
        sl = pl.ds(j * CH, CH)
        copies.append(pltpu.async_copy(ut_hbm.at[uidx.at[sl]], urows.at[sl], usem))
        copies.append(pltpu.async_copy(it_hbm.at[iidx.at[sl]], irows.at[sl], isem))
    for c in copies:
        c.wait()
    pltpu.sync_copy(urows, ue_out.at[pl.ds(base, BPW)])
    pltpu.sync_copy(irows, ie_out.at[pl.ds(base, BPW)])


def _make_gather():
    mesh = plsc.VectorSubcoreMesh(core_axis_name="c", subcore_axis_name="s")
    return pl.kernel(
        _gather_body,
        out_type=(
            jax.ShapeDtypeStruct((B, D), jnp.float32),
            jax.ShapeDtypeStruct((B, D), jnp.float32),
        ),
        mesh=mesh,
        compiler_params=pltpu.CompilerParams(use_tc_tiling_on_sc=False),
        scratch_types=[
            pltpu.VMEM((BPW,), jnp.int32),
            pltpu.VMEM((BPW,), jnp.int32),
            pltpu.VMEM((BPW, D), jnp.float32),
            pltpu.VMEM((BPW, D), jnp.float32),
            pltpu.SemaphoreType.DMA,
            pltpu.SemaphoreType.DMA,
        ],
    )


BLK = 2048


def _mlp_body(ue, ie, w1a, w1b, b1, w2, b2, w3, b3, out):
    h = jnp.dot(ue[...], w1a[...], preferred_element_type=jnp.float32)
    h = h + jnp.dot(ie[...], w1b[...], preferred_element_type=jnp.float32)
    h = jnp.maximum(h + b1[...], 0.0)
    h = jnp.dot(h, w2[...], preferred_element_type=jnp.float32) + b2[...]
    h = jnp.maximum(h, 0.0)
    o = jnp.dot(h, w3[...], preferred_element_type=jnp.float32) + b3[...]
    out[...] = jax.nn.sigmoid(o)


def _mlp(ue, ie, W1a, W1b, b1, W2, b2, W3, b3):
    grid = (B // BLK,)
    full = lambda g: (0, 0)
    return pl.pallas_call(
        _mlp_body,
        grid=grid,
        in_specs=[
            pl.BlockSpec((BLK, D), lambda g: (g, 0)),
            pl.BlockSpec((BLK, D), lambda g: (g, 0)),
            pl.BlockSpec(W1a.shape, full),
            pl.BlockSpec(W1b.shape, full),
            pl.BlockSpec(b1.shape, full),
            pl.BlockSpec(W2.shape, full),
            pl.BlockSpec(b2.shape, full),
            pl.BlockSpec(W3.shape, full),
            pl.BlockSpec(b3.shape, full),
        ],
        out_specs=pl.BlockSpec((BLK, 1), lambda g: (g, 0)),
        out_shape=jax.ShapeDtypeStruct((B, 1), jnp.float32),
    )(ue, ie, W1a, W1b, b1, W2, b2, W3, b3)


def kernel(u, i, user_table, item_table, W1, b1, W2, b2, W3, b3):
    ue, ie = _make_gather()(u, i, user_table, item_table)
    out = _mlp(ue, ie,
               W1[:D], W1[D:],
               b1.reshape(1, -1),
               W2, b2.reshape(1, -1),
               W3, b3.reshape(1, -1))
    return jnp.squeeze(out, axis=-1)
